# baseline (speedup 1.0000x reference)
import functools

import jax
import jax.numpy as jnp
from jax import lax
from jax.experimental import pallas as pl
from jax.experimental.pallas import tpu as pltpu

N = 100000
NUM_EMB = 128
EMB_DIM = 128
OP_FEAT_DIM = 140
CFG_DIM = 18
OUT_DIM = 128
MAX_NORM = 1.0

BLOCK_N = 4096
ROWS = BLOCK_N // 8
OPF_LANES = 8 * OP_FEAT_DIM
CFG_LANES = 8 * CFG_DIM
N_ROWS = N // 8


def _main_kernel(opf_ref, cfg_ref, code_ref, emb_ref, w1_ref, w2_ref,
                 w3_ref, opw_ref, cfgw_ref, b_ref, out_ref):
    rows = emb_ref[...]
    norms = jnp.sqrt(jnp.sum(rows * rows, axis=1, keepdims=True))
    scale = jnp.where(norms > MAX_NORM, MAX_NORM / (norms + 1e-7), 1.0)
    scaled = rows * (scale * opw_ref[0, 0])
    t2t = jnp.dot(w2_ref[...], scaled.T,
                  preferred_element_type=jnp.float32)

    codes = code_ref[...]
    onehot_t = (codes[None, :] ==
                lax.broadcasted_iota(jnp.int32, (NUM_EMB, BLOCK_N), 0))
    emb_t = jnp.dot(t2t, onehot_t.astype(jnp.float32),
                    preferred_element_type=jnp.float32)

    opf_t = opf_ref[...].T
    cfg_t = cfg_ref[...].T
    w1 = w1_ref[...]
    w3s = w3_ref[...] * cfgw_ref[...]
    pieces = []
    for n in range(8):
        p = jnp.dot(w1, opf_t[n * OP_FEAT_DIM:(n + 1) * OP_FEAT_DIM, :],
                    preferred_element_type=jnp.float32)
        p = p + jnp.dot(w3s, cfg_t[n * CFG_DIM:(n + 1) * CFG_DIM, :],
                        preferred_element_type=jnp.float32)
        pieces.append(p)
    ct = jnp.concatenate(pieces, axis=0)
    acc = ct.T.reshape(ROWS, 8, OUT_DIM).reshape(BLOCK_N, OUT_DIM)
    acc = acc + emb_t.T
    out_ref[...] = acc + b_ref[...]


@functools.partial(jax.jit, static_argnames=("interpret",))
def _run(op_feats, config_feats, emb_table, W, b, op_weights, config_weights,
         op_code, interpret=False):
    w1 = W[:, :OP_FEAT_DIM]
    w2 = W[:, OP_FEAT_DIM:OP_FEAT_DIM + EMB_DIM]
    w3 = W[:, OP_FEAT_DIM + EMB_DIM:]
    opw = op_weights.astype(jnp.float32).reshape(1, 1)
    cfgw = config_weights.astype(jnp.float32).reshape(1, CFG_DIM)
    codes = op_code.astype(jnp.int32).reshape(N)
    b2 = b.reshape(1, OUT_DIM)
    opf_p = op_feats.reshape(N_ROWS, OPF_LANES)
    cfg_p = config_feats.reshape(N_ROWS, CFG_LANES)

    grid = (N + BLOCK_N - 1) // BLOCK_N
    out = pl.pallas_call(
        _main_kernel,
        grid=(grid,),
        in_specs=[
            pl.BlockSpec((ROWS, OPF_LANES), lambda i: (i, 0)),
            pl.BlockSpec((ROWS, CFG_LANES), lambda i: (i, 0)),
            pl.BlockSpec((BLOCK_N,), lambda i: (i,)),
            pl.BlockSpec((NUM_EMB, EMB_DIM), lambda i: (0, 0)),
            pl.BlockSpec((OUT_DIM, OP_FEAT_DIM), lambda i: (0, 0)),
            pl.BlockSpec((OUT_DIM, EMB_DIM), lambda i: (0, 0)),
            pl.BlockSpec((OUT_DIM, CFG_DIM), lambda i: (0, 0)),
            pl.BlockSpec((1, 1), lambda i: (0, 0)),
            pl.BlockSpec((1, CFG_DIM), lambda i: (0, 0)),
            pl.BlockSpec((1, OUT_DIM), lambda i: (0, 0)),
        ],
        out_specs=pl.BlockSpec((BLOCK_N, OUT_DIM), lambda i: (i, 0)),
        out_shape=jax.ShapeDtypeStruct((N, OUT_DIM), jnp.float32),
        compiler_params=pltpu.CompilerParams(
            dimension_semantics=("arbitrary",),
        ),
        interpret=interpret,
    )(opf_p, cfg_p, codes, emb_table, w1, w2, w3, opw, cfgw, b2)
    return out


def kernel(op_feats, config_feats, emb_table, W, b, op_weights, config_weights,
           op_code):
    return _run(op_feats, config_feats, emb_table, W, b, op_weights,
                config_weights, op_code)


# speedup vs baseline: 2.3099x; 2.3099x over previous
import functools

import jax
import jax.numpy as jnp
from jax import lax
from jax.experimental import pallas as pl
from jax.experimental.pallas import tpu as pltpu
from jax.experimental.pallas import tpu_sc as plsc

N = 100000
NUM_EMB = 128
EMB_DIM = 128
OP_FEAT_DIM = 140
CFG_DIM = 18
OUT_DIM = 128
MAX_NORM = 1.0

BLOCK_N = 8192
ROWS = BLOCK_N // 8
CFG_LANES = 8 * CFG_DIM

_SC_NC = 2
_SC_NS = 16
_NW = _SC_NC * _SC_NS
_CH = 56
_CIN = _CH * 8
_RPW = 7 * _CH
_NOUT_PAD = _NW * _RPW
_N_IN_TAIL = N - (_NOUT_PAD - _CH) * 8


def _repack_sc(cfg_hbm, out_hbm, in_v, out_v):
    wid = lax.axis_index("s") * _SC_NC + lax.axis_index("c")
    lanes = lax.iota(jnp.int32, 16)

    def repack_rows(nrows):
        @pl.loop(0, nrows)
        def _row(r):
            base = r * CFG_LANES
            for s in range(CFG_LANES // 16):
                f = base + s * 16 + lanes
                vals = plsc.load_gather(in_v, [f // CFG_DIM, f % CFG_DIM])
                out_v[r, pl.ds(s * 16, 16)] = vals

    def full_chunk(out0):
        pltpu.sync_copy(cfg_hbm.at[pl.ds(out0 * 8, _CIN), :], in_v)
        repack_rows(_CH)
        pltpu.sync_copy(out_v, out_hbm.at[pl.ds(out0, _CH), :])

    for c in range(6):
        full_chunk(wid * _RPW + c * _CH)

    last0 = wid * _RPW + 6 * _CH

    @pl.when(wid < _NW - 1)
    def _():
        full_chunk(last0)

    @pl.when(wid == _NW - 1)
    def _():
        pltpu.sync_copy(cfg_hbm.at[pl.ds(N - _N_IN_TAIL, _N_IN_TAIL), :],
                        in_v.at[pl.ds(0, _N_IN_TAIL), :])
        repack_rows(_CH)
        pltpu.sync_copy(out_v, out_hbm.at[pl.ds(_NOUT_PAD - _CH, _CH), :])


def _main_kernel(opf_ref, cfgp_ref, code_ref, emb_ref, w1t_ref, w2_ref,
                 w3_ref, opw_ref, cfgw_ref, b_ref, out_ref):
    rows = emb_ref[...]
    norms = jnp.sqrt(jnp.sum(rows * rows, axis=1, keepdims=True))
    scale = jnp.where(norms > MAX_NORM, MAX_NORM / (norms + 1e-7), 1.0)
    scaled = rows * (scale * opw_ref[0, 0])
    t2t = jnp.dot(w2_ref[...], scaled.T,
                  preferred_element_type=jnp.float32)

    codes = code_ref[...]
    onehot_t = (codes[None, :] ==
                lax.broadcasted_iota(jnp.int32, (NUM_EMB, BLOCK_N), 0))
    emb_t = jnp.dot(t2t, onehot_t.astype(jnp.float32),
                    preferred_element_type=jnp.float32)

    cfg_t = cfgp_ref[...].T
    w3s = w3_ref[...] * cfgw_ref[...]
    pieces = [
        jnp.dot(w3s, cfg_t[n * CFG_DIM:(n + 1) * CFG_DIM, :],
                preferred_element_type=jnp.float32)
        for n in range(8)
    ]
    ct = jnp.concatenate(pieces, axis=0)
    cfg_contrib = ct.T.reshape(ROWS, 8, OUT_DIM).reshape(BLOCK_N, OUT_DIM)

    acc = jnp.dot(opf_ref[...], w1t_ref[...],
                  preferred_element_type=jnp.float32)
    acc = acc + emb_t.T + cfg_contrib
    out_ref[...] = acc + b_ref[...]


@functools.partial(jax.jit, static_argnames=("interpret",))
def _run(op_feats, config_feats, emb_table, W, b, op_weights, config_weights,
         op_code, interpret=False):
    w1t = W[:, :OP_FEAT_DIM].T
    w2 = W[:, OP_FEAT_DIM:OP_FEAT_DIM + EMB_DIM]
    w3 = W[:, OP_FEAT_DIM + EMB_DIM:]
    opw = op_weights.astype(jnp.float32).reshape(1, 1)
    cfgw = config_weights.astype(jnp.float32).reshape(1, CFG_DIM)
    codes = op_code.astype(jnp.int32).reshape(N)
    b2 = b.reshape(1, OUT_DIM)

    repack = pl.kernel(
        _repack_sc,
        out_type=jax.ShapeDtypeStruct((_NOUT_PAD, CFG_LANES), jnp.float32),
        mesh=plsc.VectorSubcoreMesh(core_axis_name="c", subcore_axis_name="s",
                                    num_cores=_SC_NC, num_subcores=_SC_NS),
        scratch_types=[
            pltpu.VMEM((_CIN, CFG_DIM), jnp.float32),
            pltpu.VMEM((_CH, CFG_LANES), jnp.float32),
        ],
        compiler_params=pltpu.CompilerParams(use_tc_tiling_on_sc=False,
                                             needs_layout_passes=False),
    )
    cfgp = repack(config_feats)

    grid = (N + BLOCK_N - 1) // BLOCK_N
    out = pl.pallas_call(
        _main_kernel,
        grid=(grid,),
        in_specs=[
            pl.BlockSpec((BLOCK_N, OP_FEAT_DIM), lambda i: (i, 0)),
            pl.BlockSpec((ROWS, CFG_LANES), lambda i: (i, 0)),
            pl.BlockSpec((BLOCK_N,), lambda i: (i,)),
            pl.BlockSpec((NUM_EMB, EMB_DIM), lambda i: (0, 0)),
            pl.BlockSpec((OP_FEAT_DIM, OUT_DIM), lambda i: (0, 0)),
            pl.BlockSpec((OUT_DIM, EMB_DIM), lambda i: (0, 0)),
            pl.BlockSpec((OUT_DIM, CFG_DIM), lambda i: (0, 0)),
            pl.BlockSpec((1, 1), lambda i: (0, 0)),
            pl.BlockSpec((1, CFG_DIM), lambda i: (0, 0)),
            pl.BlockSpec((1, OUT_DIM), lambda i: (0, 0)),
        ],
        out_specs=pl.BlockSpec((BLOCK_N, OUT_DIM), lambda i: (i, 0)),
        out_shape=jax.ShapeDtypeStruct((N, OUT_DIM), jnp.float32),
        compiler_params=pltpu.CompilerParams(
            dimension_semantics=("arbitrary",),
        ),
        interpret=interpret,
    )(op_feats, cfgp, codes, emb_table, w1t, w2, w3, opw, cfgw, b2)
    return out


def kernel(op_feats, config_feats, emb_table, W, b, op_weights, config_weights,
           op_code):
    return _run(op_feats, config_feats, emb_table, W, b, op_weights,
                config_weights, op_code)


# speedup vs baseline: 3.4890x; 1.5104x over previous
import functools

import jax
import jax.numpy as jnp
from jax import lax
from jax.experimental import pallas as pl
from jax.experimental.pallas import tpu as pltpu

N = 100000
NUM_EMB = 128
EMB_DIM = 128
OP_FEAT_DIM = 140
CFG_DIM = 18
OUT_DIM = 128
MAX_NORM = 1.0

BLOCK_N = 8192


def _main_kernel(opf_ref, cfg_ref, code_ref, emb_ref, w1t_ref, w2_ref,
                 w3t_ref, opw_ref, cfgw_ref, b_ref, out_ref):
    rows = emb_ref[...]
    norms = jnp.sqrt(jnp.sum(rows * rows, axis=1, keepdims=True))
    scale = jnp.where(norms > MAX_NORM, MAX_NORM / (norms + 1e-7), 1.0)
    scaled = rows * (scale * opw_ref[0, 0])
    t2t = jnp.dot(w2_ref[...], scaled.T,
                  preferred_element_type=jnp.float32)

    codes = code_ref[...]
    onehot_t = (codes[None, :] ==
                lax.broadcasted_iota(jnp.int32, (NUM_EMB, BLOCK_N), 0))
    emb_t = jnp.dot(t2t, onehot_t.astype(jnp.float32),
                    preferred_element_type=jnp.float32)
    acc = jnp.dot(opf_ref[...], w1t_ref[...], preferred_element_type=jnp.float32)
    acc = acc + emb_t.T
    acc = acc + jnp.dot(cfg_ref[...] * cfgw_ref[...], w3t_ref[...],
                        preferred_element_type=jnp.float32)
    out_ref[...] = acc + b_ref[...]


@functools.partial(jax.jit, static_argnames=("interpret",))
def _run(op_feats, config_feats, emb_table, W, b, op_weights, config_weights,
         op_code, interpret=False):
    w1t = W[:, :OP_FEAT_DIM].T
    w2 = W[:, OP_FEAT_DIM:OP_FEAT_DIM + EMB_DIM]
    w3t = W[:, OP_FEAT_DIM + EMB_DIM:].T
    opw = op_weights.astype(jnp.float32).reshape(1, 1)
    cfgw = config_weights.astype(jnp.float32).reshape(1, CFG_DIM)
    codes = op_code.astype(jnp.int32).reshape(N)
    b2 = b.reshape(1, OUT_DIM)

    grid = (N + BLOCK_N - 1) // BLOCK_N
    out = pl.pallas_call(
        _main_kernel,
        grid=(grid,),
        in_specs=[
            pl.BlockSpec((BLOCK_N, OP_FEAT_DIM), lambda i: (i, 0)),
            pl.BlockSpec((BLOCK_N, CFG_DIM), lambda i: (i, 0)),
            pl.BlockSpec((BLOCK_N,), lambda i: (i,)),
            pl.BlockSpec((NUM_EMB, EMB_DIM), lambda i: (0, 0)),
            pl.BlockSpec((OP_FEAT_DIM, OUT_DIM), lambda i: (0, 0)),
            pl.BlockSpec((OUT_DIM, EMB_DIM), lambda i: (0, 0)),
            pl.BlockSpec((CFG_DIM, OUT_DIM), lambda i: (0, 0)),
            pl.BlockSpec((1, 1), lambda i: (0, 0)),
            pl.BlockSpec((1, CFG_DIM), lambda i: (0, 0)),
            pl.BlockSpec((1, OUT_DIM), lambda i: (0, 0)),
        ],
        out_specs=pl.BlockSpec((BLOCK_N, OUT_DIM), lambda i: (i, 0)),
        out_shape=jax.ShapeDtypeStruct((N, OUT_DIM), jnp.float32),
        compiler_params=pltpu.CompilerParams(
            dimension_semantics=("arbitrary",),
        ),
        interpret=interpret,
    )(op_feats, config_feats, codes, emb_table, w1t, w2, w3t, opw, cfgw, b2)
    return out


def kernel(op_feats, config_feats, emb_table, W, b, op_weights, config_weights,
           op_code):
    return _run(op_feats, config_feats, emb_table, W, b, op_weights,
                config_weights, op_code)
